# baseline (device time: 92215 ns/iter reference)
import functools

import jax
import jax.numpy as jnp
from jax import lax
from jax.experimental import pallas as pl
from jax.experimental.pallas import tpu as pltpu

N_DEV = 8
B = 64
D = 2048
H_SHARD = 4096
D_CHUNK = 512
H_CHUNK = 1024
N_CHUNKS = 4
SLOTS = 3
_R = B // N_DEV


def _start_ar(my, src_f32_ref, xb_ref, rs_ref, send_sems, recv_sems):
    xb_ref[...] = src_f32_ref[...].astype(jnp.bfloat16)
    rdmas = []
    for r in range(1, N_DEV):
        partner = jnp.bitwise_xor(my, r)
        rdma = pltpu.make_async_remote_copy(
            src_ref=xb_ref.at[pl.ds(partner * _R, _R)],
            dst_ref=rs_ref.at[r - 1],
            send_sem=send_sems.at[r - 1],
            recv_sem=recv_sems.at[r - 1],
            device_id=(partner,),
            device_id_type=pl.DeviceIdType.MESH,
        )
        rdma.start()
        rdmas.append(rdma)
    return rdmas


def _finish_rs(my, rs_rdmas, src_f32_ref, rs_ref):
    for rdma in rs_rdmas:
        rdma.wait()
    acc = src_f32_ref[pl.ds(my * _R, _R), :]
    for r in range(1, N_DEV):
        acc = acc + rs_ref[r - 1].astype(jnp.float32)
    return acc


def _finish_ag(my, acc, x_ref, xb_ref, send_sems, recv_sems):
    xb_ref[pl.ds(my * _R, _R), :] = acc.astype(jnp.bfloat16)
    rdmas = []
    for r in range(1, N_DEV):
        partner = jnp.bitwise_xor(my, r)
        rdma = pltpu.make_async_remote_copy(
            src_ref=xb_ref.at[pl.ds(my * _R, _R)],
            dst_ref=xb_ref.at[pl.ds(my * _R, _R)],
            send_sem=send_sems.at[N_DEV - 1 + r - 1],
            recv_sem=recv_sems.at[N_DEV - 1 + r - 1],
            device_id=(partner,),
            device_id_type=pl.DeviceIdType.MESH,
        )
        rdma.start()
        rdmas.append(rdma)
    for rdma in rdmas:
        rdma.wait()
    x_ref[...] = xb_ref[...].astype(jnp.float32)
    x_ref[pl.ds(my * _R, _R), :] = acc


def _layer_body(do_ar, p_ref, win_hbm, wout_hbm, out_ref,
                x_ref, xb_ref, rs_ref, h_ref, win_buf, wout_buf,
                send_sems, recv_sems, win_sems, wout_sems):
    my = lax.axis_index("i")

    def issue_win(c):
        s = c % SLOTS
        cp = pltpu.make_async_copy(
            win_hbm.at[pl.ds(c * D_CHUNK, D_CHUNK), :], win_buf.at[s],
            win_sems.at[s],
        )
        cp.start()
        return cp

    def issue_wout(c):
        s = c % SLOTS
        cp = pltpu.make_async_copy(
            wout_hbm.at[pl.ds(c * H_CHUNK, H_CHUNK), :], wout_buf.at[s],
            wout_sems.at[s],
        )
        cp.start()
        return cp

    pend_win = {c: issue_win(c) for c in range(SLOTS)}
    pend_wout = {c: issue_wout(c) for c in range(SLOTS)}

    if do_ar:
        barrier_sem = pltpu.get_barrier_semaphore()
        for r in range(1, N_DEV):
            pl.semaphore_signal(
                barrier_sem, inc=1,
                device_id=(jnp.bitwise_xor(my, r),),
                device_id_type=pl.DeviceIdType.MESH,
            )
        pl.semaphore_wait(barrier_sem, N_DEV - 1)

        rs_rdmas = _start_ar(my, p_ref, xb_ref, rs_ref, send_sems, recv_sems)
        acc = _finish_rs(my, rs_rdmas, p_ref, rs_ref)
        _finish_ag(my, acc, x_ref, xb_ref, send_sems, recv_sems)
        x = x_ref[...]
    else:
        x = p_ref[...]

    h_ref[...] = jnp.zeros_like(h_ref)
    for c in range(N_CHUNKS):
        pend_win.pop(c).wait()
        h_ref[...] += jnp.dot(
            x[:, c * D_CHUNK:(c + 1) * D_CHUNK], win_buf[c % SLOTS],
            preferred_element_type=jnp.float32,
        )
        if c + SLOTS < N_CHUNKS:
            pend_win[c + SLOTS] = issue_win(c + SLOTS)
    h = jnp.maximum(h_ref[...], 0.0)

    out_ref[...] = jnp.zeros_like(out_ref)
    for c in range(N_CHUNKS):
        pend_wout.pop(c).wait()
        out_ref[...] += jnp.dot(
            h[:, c * H_CHUNK:(c + 1) * H_CHUNK], wout_buf[c % SLOTS],
            preferred_element_type=jnp.float32,
        )
        if c + SLOTS < N_CHUNKS:
            pend_wout[c + SLOTS] = issue_wout(c + SLOTS)


def _layer(x, win, wout, *, collective_id=None):
    do_ar = collective_id is not None
    return pl.pallas_call(
        functools.partial(_layer_body, do_ar),
        in_specs=[
            pl.BlockSpec(memory_space=pltpu.VMEM),
            pl.BlockSpec(memory_space=pl.ANY),
            pl.BlockSpec(memory_space=pl.ANY),
        ],
        out_specs=pl.BlockSpec(memory_space=pltpu.VMEM),
        out_shape=jax.ShapeDtypeStruct((B, D), jnp.float32),
        scratch_shapes=[
            pltpu.VMEM((B, D), jnp.float32),
            pltpu.VMEM((B, D), jnp.bfloat16),
            pltpu.VMEM((N_DEV - 1, _R, D), jnp.bfloat16),
            pltpu.VMEM((B, H_SHARD), jnp.float32),
            pltpu.VMEM((SLOTS, D_CHUNK, H_SHARD), jnp.float32),
            pltpu.VMEM((SLOTS, H_CHUNK, D), jnp.float32),
            pltpu.SemaphoreType.DMA((2 * (N_DEV - 1),)),
            pltpu.SemaphoreType.DMA((2 * (N_DEV - 1),)),
            pltpu.SemaphoreType.DMA((SLOTS,)),
            pltpu.SemaphoreType.DMA((SLOTS,)),
        ],
        compiler_params=pltpu.CompilerParams(
            collective_id=collective_id,
            vmem_limit_bytes=60 * 1024 * 1024,
        ),
    )(x, win, wout)


def _rs_body(p_ref, out_ref, xb_ref, rs_ref, send_sems, recv_sems):
    my = lax.axis_index("i")
    barrier_sem = pltpu.get_barrier_semaphore()
    for r in range(1, N_DEV):
        pl.semaphore_signal(
            barrier_sem, inc=1,
            device_id=(jnp.bitwise_xor(my, r),),
            device_id_type=pl.DeviceIdType.MESH,
        )
    pl.semaphore_wait(barrier_sem, N_DEV - 1)
    rs_rdmas = _start_ar(my, p_ref, xb_ref, rs_ref, send_sems, recv_sems)
    out_ref[...] = _finish_rs(my, rs_rdmas, p_ref, rs_ref)


def _reduce_scatter(p, *, collective_id):
    return pl.pallas_call(
        _rs_body,
        out_shape=jax.ShapeDtypeStruct((B // N_DEV, D), jnp.float32),
        in_specs=[pl.BlockSpec(memory_space=pltpu.VMEM)],
        out_specs=pl.BlockSpec(memory_space=pltpu.VMEM),
        scratch_shapes=[
            pltpu.VMEM((B, D), jnp.bfloat16),
            pltpu.VMEM((N_DEV - 1, _R, D), jnp.bfloat16),
            pltpu.SemaphoreType.DMA((2 * (N_DEV - 1),)),
            pltpu.SemaphoreType.DMA((2 * (N_DEV - 1),)),
        ],
        compiler_params=pltpu.CompilerParams(collective_id=collective_id),
    )(p)


def kernel(x, Win0, Wout0, Win1, Wout1, Win2, Wout2):
    p0 = _layer(x, Win0, Wout0)
    p1 = _layer(p0, Win1, Wout1, collective_id=0)
    p2 = _layer(p1, Win2, Wout2, collective_id=1)
    return _reduce_scatter(p2, collective_id=2)


# device time: 86390 ns/iter; 1.0674x vs baseline; 1.0674x over previous
import functools

import jax
import jax.numpy as jnp
from jax import lax
from jax.experimental import pallas as pl
from jax.experimental.pallas import tpu as pltpu

N_DEV = 8
B = 64
D = 2048
H_SHARD = 4096
H_CHUNK = 512
N_CHUNKS = H_SHARD // H_CHUNK
SLOTS = 6
_R = B // N_DEV


def _start_ar(my, src_f32_ref, xb_ref, rs_ref, send_sems, recv_sems):
    xb_ref[...] = src_f32_ref[...].astype(jnp.bfloat16)
    rdmas = []
    for r in range(1, N_DEV):
        partner = jnp.bitwise_xor(my, r)
        rdma = pltpu.make_async_remote_copy(
            src_ref=xb_ref.at[pl.ds(partner * _R, _R)],
            dst_ref=rs_ref.at[r - 1],
            send_sem=send_sems.at[r - 1],
            recv_sem=recv_sems.at[r - 1],
            device_id=(partner,),
            device_id_type=pl.DeviceIdType.MESH,
        )
        rdma.start()
        rdmas.append(rdma)
    return rdmas


def _finish_rs(my, rs_rdmas, src_f32_ref, rs_ref):
    for rdma in rs_rdmas:
        rdma.wait()
    acc = src_f32_ref[pl.ds(my * _R, _R), :]
    for r in range(1, N_DEV):
        acc = acc + rs_ref[r - 1].astype(jnp.float32)
    return acc


def _finish_ag(my, acc, x_ref, xb_ref, send_sems, recv_sems):
    xb_ref[pl.ds(my * _R, _R), :] = acc.astype(jnp.bfloat16)
    rdmas = []
    for r in range(1, N_DEV):
        partner = jnp.bitwise_xor(my, r)
        rdma = pltpu.make_async_remote_copy(
            src_ref=xb_ref.at[pl.ds(my * _R, _R)],
            dst_ref=xb_ref.at[pl.ds(my * _R, _R)],
            send_sem=send_sems.at[N_DEV - 1 + r - 1],
            recv_sem=recv_sems.at[N_DEV - 1 + r - 1],
            device_id=(partner,),
            device_id_type=pl.DeviceIdType.MESH,
        )
        rdma.start()
        rdmas.append(rdma)
    for rdma in rdmas:
        rdma.wait()
    x_ref[...] = xb_ref[...].astype(jnp.float32)
    x_ref[pl.ds(my * _R, _R), :] = acc


def _layer_body(do_ar, p_ref, win_hbm, wout_hbm, out_ref,
                x_ref, xb_ref, rs_ref, win_buf, wout_buf,
                send_sems, recv_sems, win_sems, wout_sems):
    my = lax.axis_index("i")

    def issue_w(c):
        s = c % SLOTS
        cw = pltpu.make_async_copy(
            win_hbm.at[:, pl.ds(c * H_CHUNK, H_CHUNK)], win_buf.at[s],
            win_sems.at[s],
        )
        cw.start()
        co = pltpu.make_async_copy(
            wout_hbm.at[pl.ds(c * H_CHUNK, H_CHUNK), :], wout_buf.at[s],
            wout_sems.at[s],
        )
        co.start()
        return cw, co

    pending = {c: issue_w(c) for c in range(min(SLOTS, N_CHUNKS))}

    if do_ar:
        barrier_sem = pltpu.get_barrier_semaphore()
        for r in range(1, N_DEV):
            pl.semaphore_signal(
                barrier_sem, inc=1,
                device_id=(jnp.bitwise_xor(my, r),),
                device_id_type=pl.DeviceIdType.MESH,
            )
        pl.semaphore_wait(barrier_sem, N_DEV - 1)

        rs_rdmas = _start_ar(my, p_ref, xb_ref, rs_ref, send_sems, recv_sems)
        acc = _finish_rs(my, rs_rdmas, p_ref, rs_ref)
        _finish_ag(my, acc, x_ref, xb_ref, send_sems, recv_sems)
        x = x_ref[...]
    else:
        x = p_ref[...]

    out_ref[...] = jnp.zeros_like(out_ref)
    for c in range(N_CHUNKS):
        s = c % SLOTS
        cw, co = pending.pop(c)
        cw.wait()
        co.wait()
        h = jnp.dot(x, win_buf[s], preferred_element_type=jnp.float32)
        h = jnp.maximum(h, 0.0)
        out_ref[...] += jnp.dot(h, wout_buf[s], preferred_element_type=jnp.float32)
        if c + SLOTS < N_CHUNKS:
            pending[c + SLOTS] = issue_w(c + SLOTS)


def _layer(x, win, wout, *, collective_id=None):
    do_ar = collective_id is not None
    return pl.pallas_call(
        functools.partial(_layer_body, do_ar),
        in_specs=[
            pl.BlockSpec(memory_space=pltpu.VMEM),
            pl.BlockSpec(memory_space=pl.ANY),
            pl.BlockSpec(memory_space=pl.ANY),
        ],
        out_specs=pl.BlockSpec(memory_space=pltpu.VMEM),
        out_shape=jax.ShapeDtypeStruct((B, D), jnp.float32),
        scratch_shapes=[
            pltpu.VMEM((B, D), jnp.float32),
            pltpu.VMEM((B, D), jnp.bfloat16),
            pltpu.VMEM((N_DEV - 1, _R, D), jnp.bfloat16),
            pltpu.VMEM((SLOTS, D, H_CHUNK), jnp.float32),
            pltpu.VMEM((SLOTS, H_CHUNK, D), jnp.float32),
            pltpu.SemaphoreType.DMA((2 * (N_DEV - 1),)),
            pltpu.SemaphoreType.DMA((2 * (N_DEV - 1),)),
            pltpu.SemaphoreType.DMA((SLOTS,)),
            pltpu.SemaphoreType.DMA((SLOTS,)),
        ],
        compiler_params=pltpu.CompilerParams(
            collective_id=collective_id,
            vmem_limit_bytes=60 * 1024 * 1024,
        ),
    )(x, win, wout)


def _rs_body(p_ref, out_ref, xb_ref, rs_ref, send_sems, recv_sems):
    my = lax.axis_index("i")
    barrier_sem = pltpu.get_barrier_semaphore()
    for r in range(1, N_DEV):
        pl.semaphore_signal(
            barrier_sem, inc=1,
            device_id=(jnp.bitwise_xor(my, r),),
            device_id_type=pl.DeviceIdType.MESH,
        )
    pl.semaphore_wait(barrier_sem, N_DEV - 1)
    rs_rdmas = _start_ar(my, p_ref, xb_ref, rs_ref, send_sems, recv_sems)
    out_ref[...] = _finish_rs(my, rs_rdmas, p_ref, rs_ref)


def _reduce_scatter(p, *, collective_id):
    return pl.pallas_call(
        _rs_body,
        out_shape=jax.ShapeDtypeStruct((B // N_DEV, D), jnp.float32),
        in_specs=[pl.BlockSpec(memory_space=pltpu.VMEM)],
        out_specs=pl.BlockSpec(memory_space=pltpu.VMEM),
        scratch_shapes=[
            pltpu.VMEM((B, D), jnp.bfloat16),
            pltpu.VMEM((N_DEV - 1, _R, D), jnp.bfloat16),
            pltpu.SemaphoreType.DMA((2 * (N_DEV - 1),)),
            pltpu.SemaphoreType.DMA((2 * (N_DEV - 1),)),
        ],
        compiler_params=pltpu.CompilerParams(collective_id=collective_id),
    )(p)


def kernel(x, Win0, Wout0, Win1, Wout1, Win2, Wout2):
    p0 = _layer(x, Win0, Wout0)
    p1 = _layer(p0, Win1, Wout1, collective_id=0)
    p2 = _layer(p1, Win2, Wout2, collective_id=1)
    return _reduce_scatter(p2, collective_id=2)


# device time: 82915 ns/iter; 1.1122x vs baseline; 1.0419x over previous
import functools

import jax
import jax.numpy as jnp
from jax import lax
from jax.experimental import pallas as pl
from jax.experimental.pallas import tpu as pltpu

N_DEV = 8
B = 64
D = 2048
H_SHARD = 4096
H_CHUNK = 512
N_CHUNKS = H_SHARD // H_CHUNK
SLOTS = 6
_R = B // N_DEV


def _layer0_body(x_ref, win_ref, wout_ref, out_ref, acc_ref):
    c = pl.program_id(0)

    @pl.when(c == 0)
    def _():
        acc_ref[...] = jnp.zeros_like(acc_ref)

    h = jnp.dot(x_ref[...], win_ref[...], preferred_element_type=jnp.float32)
    h = jnp.maximum(h, 0.0)
    acc_ref[...] += jnp.dot(h, wout_ref[...], preferred_element_type=jnp.float32)

    @pl.when(c == pl.num_programs(0) - 1)
    def _():
        out_ref[...] = acc_ref[...]


def _layer0(x, win, wout):
    return pl.pallas_call(
        _layer0_body,
        grid=(N_CHUNKS,),
        in_specs=[
            pl.BlockSpec((B, D), lambda c: (0, 0)),
            pl.BlockSpec((D, H_CHUNK), lambda c: (0, c)),
            pl.BlockSpec((H_CHUNK, D), lambda c: (c, 0)),
        ],
        out_specs=pl.BlockSpec((B, D), lambda c: (0, 0)),
        out_shape=jax.ShapeDtypeStruct((B, D), jnp.float32),
        scratch_shapes=[pltpu.VMEM((B, D), jnp.float32)],
    )(x, win, wout)


def _start_ar(my, src_f32_ref, xb_ref, rs_ref, send_sems, recv_sems):
    xb_ref[...] = src_f32_ref[...].astype(jnp.bfloat16)
    rdmas = []
    for r in range(1, N_DEV):
        partner = jnp.bitwise_xor(my, r)
        rdma = pltpu.make_async_remote_copy(
            src_ref=xb_ref.at[pl.ds(partner * _R, _R)],
            dst_ref=rs_ref.at[r - 1],
            send_sem=send_sems.at[r - 1],
            recv_sem=recv_sems.at[r - 1],
            device_id=(partner,),
            device_id_type=pl.DeviceIdType.MESH,
        )
        rdma.start()
        rdmas.append(rdma)
    return rdmas


def _finish_rs(my, rs_rdmas, src_f32_ref, rs_ref):
    for rdma in rs_rdmas:
        rdma.wait()
    acc = src_f32_ref[pl.ds(my * _R, _R), :]
    for r in range(1, N_DEV):
        acc = acc + rs_ref[r - 1].astype(jnp.float32)
    return acc


def _finish_ag(my, acc, x_ref, xb_ref, send_sems, recv_sems):
    xb_ref[pl.ds(my * _R, _R), :] = acc.astype(jnp.bfloat16)
    rdmas = []
    for r in range(1, N_DEV):
        partner = jnp.bitwise_xor(my, r)
        rdma = pltpu.make_async_remote_copy(
            src_ref=xb_ref.at[pl.ds(my * _R, _R)],
            dst_ref=xb_ref.at[pl.ds(my * _R, _R)],
            send_sem=send_sems.at[N_DEV - 1 + r - 1],
            recv_sem=recv_sems.at[N_DEV - 1 + r - 1],
            device_id=(partner,),
            device_id_type=pl.DeviceIdType.MESH,
        )
        rdma.start()
        rdmas.append(rdma)
    for rdma in rdmas:
        rdma.wait()
    x_ref[...] = xb_ref[...].astype(jnp.float32)
    x_ref[pl.ds(my * _R, _R), :] = acc


def _layer_body(do_ar, p_ref, win_hbm, wout_hbm, out_ref,
                x_ref, xb_ref, rs_ref, win_buf, wout_buf,
                send_sems, recv_sems, win_sems, wout_sems):
    my = lax.axis_index("i")

    def issue_w(c):
        s = c % SLOTS
        cw = pltpu.make_async_copy(
            win_hbm.at[:, pl.ds(c * H_CHUNK, H_CHUNK)], win_buf.at[s],
            win_sems.at[s],
        )
        cw.start()
        co = pltpu.make_async_copy(
            wout_hbm.at[pl.ds(c * H_CHUNK, H_CHUNK), :], wout_buf.at[s],
            wout_sems.at[s],
        )
        co.start()
        return cw, co

    pending = {c: issue_w(c) for c in range(min(SLOTS, N_CHUNKS))}

    if do_ar:
        barrier_sem = pltpu.get_barrier_semaphore()
        for r in range(1, N_DEV):
            pl.semaphore_signal(
                barrier_sem, inc=1,
                device_id=(jnp.bitwise_xor(my, r),),
                device_id_type=pl.DeviceIdType.MESH,
            )
        pl.semaphore_wait(barrier_sem, N_DEV - 1)

        rs_rdmas = _start_ar(my, p_ref, xb_ref, rs_ref, send_sems, recv_sems)
        acc = _finish_rs(my, rs_rdmas, p_ref, rs_ref)
        _finish_ag(my, acc, x_ref, xb_ref, send_sems, recv_sems)
        x = x_ref[...]
    else:
        x = p_ref[...]

    out_ref[...] = jnp.zeros_like(out_ref)
    for c in range(N_CHUNKS):
        s = c % SLOTS
        cw, co = pending.pop(c)
        cw.wait()
        co.wait()
        h = jnp.dot(x, win_buf[s], preferred_element_type=jnp.float32)
        h = jnp.maximum(h, 0.0)
        out_ref[...] += jnp.dot(h, wout_buf[s], preferred_element_type=jnp.float32)
        if c + SLOTS < N_CHUNKS:
            pending[c + SLOTS] = issue_w(c + SLOTS)


def _layer(x, win, wout, *, collective_id=None):
    do_ar = collective_id is not None
    return pl.pallas_call(
        functools.partial(_layer_body, do_ar),
        in_specs=[
            pl.BlockSpec(memory_space=pltpu.VMEM),
            pl.BlockSpec(memory_space=pl.ANY),
            pl.BlockSpec(memory_space=pl.ANY),
        ],
        out_specs=pl.BlockSpec(memory_space=pltpu.VMEM),
        out_shape=jax.ShapeDtypeStruct((B, D), jnp.float32),
        scratch_shapes=[
            pltpu.VMEM((B, D), jnp.float32),
            pltpu.VMEM((B, D), jnp.bfloat16),
            pltpu.VMEM((N_DEV - 1, _R, D), jnp.bfloat16),
            pltpu.VMEM((SLOTS, D, H_CHUNK), jnp.float32),
            pltpu.VMEM((SLOTS, H_CHUNK, D), jnp.float32),
            pltpu.SemaphoreType.DMA((2 * (N_DEV - 1),)),
            pltpu.SemaphoreType.DMA((2 * (N_DEV - 1),)),
            pltpu.SemaphoreType.DMA((SLOTS,)),
            pltpu.SemaphoreType.DMA((SLOTS,)),
        ],
        compiler_params=pltpu.CompilerParams(
            collective_id=collective_id,
            vmem_limit_bytes=60 * 1024 * 1024,
        ),
    )(x, win, wout)


def _rs_body(p_ref, out_ref, xb_ref, rs_ref, send_sems, recv_sems):
    my = lax.axis_index("i")
    barrier_sem = pltpu.get_barrier_semaphore()
    for r in range(1, N_DEV):
        pl.semaphore_signal(
            barrier_sem, inc=1,
            device_id=(jnp.bitwise_xor(my, r),),
            device_id_type=pl.DeviceIdType.MESH,
        )
    pl.semaphore_wait(barrier_sem, N_DEV - 1)
    rs_rdmas = _start_ar(my, p_ref, xb_ref, rs_ref, send_sems, recv_sems)
    out_ref[...] = _finish_rs(my, rs_rdmas, p_ref, rs_ref)


def _reduce_scatter(p, *, collective_id):
    return pl.pallas_call(
        _rs_body,
        out_shape=jax.ShapeDtypeStruct((B // N_DEV, D), jnp.float32),
        in_specs=[pl.BlockSpec(memory_space=pltpu.VMEM)],
        out_specs=pl.BlockSpec(memory_space=pltpu.VMEM),
        scratch_shapes=[
            pltpu.VMEM((B, D), jnp.bfloat16),
            pltpu.VMEM((N_DEV - 1, _R, D), jnp.bfloat16),
            pltpu.SemaphoreType.DMA((2 * (N_DEV - 1),)),
            pltpu.SemaphoreType.DMA((2 * (N_DEV - 1),)),
        ],
        compiler_params=pltpu.CompilerParams(collective_id=collective_id),
    )(p)


def kernel(x, Win0, Wout0, Win1, Wout1, Win2, Wout2):
    p0 = _layer0(x, Win0, Wout0)
    p1 = _layer(p0, Win1, Wout1, collective_id=0)
    p2 = _layer(p1, Win2, Wout2, collective_id=1)
    return _reduce_scatter(p2, collective_id=2)


# device time: 81877 ns/iter; 1.1263x vs baseline; 1.0127x over previous
import functools

import jax
import jax.numpy as jnp
from jax import lax
from jax.experimental import pallas as pl
from jax.experimental.pallas import tpu as pltpu

N_DEV = 8
B = 64
D = 2048
H_SHARD = 4096
H_CHUNK = 512
N_CHUNKS = H_SHARD // H_CHUNK
SLOTS = 6
_R = B // N_DEV


def _layer0_body(x_ref, win_ref, wout_ref, out_ref, acc_ref):
    c = pl.program_id(0)

    @pl.when(c == 0)
    def _():
        acc_ref[...] = jnp.zeros_like(acc_ref)

    h = jnp.dot(x_ref[...], win_ref[...], preferred_element_type=jnp.float32)
    h = jnp.maximum(h, 0.0)
    acc_ref[...] += jnp.dot(h, wout_ref[...], preferred_element_type=jnp.float32)

    @pl.when(c == pl.num_programs(0) - 1)
    def _():
        out_ref[...] = acc_ref[...]


def _layer0(x, win, wout):
    return pl.pallas_call(
        _layer0_body,
        grid=(N_CHUNKS,),
        in_specs=[
            pl.BlockSpec((B, D), lambda c: (0, 0)),
            pl.BlockSpec((D, H_CHUNK), lambda c: (0, c)),
            pl.BlockSpec((H_CHUNK, D), lambda c: (c, 0)),
        ],
        out_specs=pl.BlockSpec((B, D), lambda c: (0, 0)),
        out_shape=jax.ShapeDtypeStruct((B, D), jnp.float32),
        scratch_shapes=[pltpu.VMEM((B, D), jnp.float32)],
    )(x, win, wout)


def _start_ar(my, src_f32_ref, xb_ref, rs_ref, send_sems, recv_sems, base=0):
    xb_ref[...] = src_f32_ref[...].astype(jnp.bfloat16)
    rdmas = []
    for r in range(1, N_DEV):
        partner = jnp.bitwise_xor(my, r)
        rdma = pltpu.make_async_remote_copy(
            src_ref=xb_ref.at[pl.ds(partner * _R, _R)],
            dst_ref=rs_ref.at[r - 1],
            send_sem=send_sems.at[base + r - 1],
            recv_sem=recv_sems.at[base + r - 1],
            device_id=(partner,),
            device_id_type=pl.DeviceIdType.MESH,
        )
        rdma.start()
        rdmas.append(rdma)
    return rdmas


def _finish_rs(my, rs_rdmas, src_f32_ref, rs_ref):
    for rdma in rs_rdmas:
        rdma.wait()
    acc = src_f32_ref[pl.ds(my * _R, _R), :]
    for r in range(1, N_DEV):
        acc = acc + rs_ref[r - 1].astype(jnp.float32)
    return acc


def _finish_ag(my, acc, x_ref, xb_ref, send_sems, recv_sems):
    xb_ref[pl.ds(my * _R, _R), :] = acc.astype(jnp.bfloat16)
    rdmas = []
    for r in range(1, N_DEV):
        partner = jnp.bitwise_xor(my, r)
        rdma = pltpu.make_async_remote_copy(
            src_ref=xb_ref.at[pl.ds(my * _R, _R)],
            dst_ref=xb_ref.at[pl.ds(my * _R, _R)],
            send_sem=send_sems.at[N_DEV - 1 + r - 1],
            recv_sem=recv_sems.at[N_DEV - 1 + r - 1],
            device_id=(partner,),
            device_id_type=pl.DeviceIdType.MESH,
        )
        rdma.start()
        rdmas.append(rdma)
    for rdma in rdmas:
        rdma.wait()
    x_ref[...] = xb_ref[...].astype(jnp.float32)
    x_ref[pl.ds(my * _R, _R), :] = acc


def _layer_body(scatter, p_ref, win_hbm, wout_hbm, out_ref,
                x_ref, xb_ref, rs_ref, rs2_ref, pacc_ref, win_buf, wout_buf,
                send_sems, recv_sems, win_sems, wout_sems):
    my = lax.axis_index("i")

    def issue_w(c):
        s = c % SLOTS
        cw = pltpu.make_async_copy(
            win_hbm.at[:, pl.ds(c * H_CHUNK, H_CHUNK)], win_buf.at[s],
            win_sems.at[s],
        )
        cw.start()
        co = pltpu.make_async_copy(
            wout_hbm.at[pl.ds(c * H_CHUNK, H_CHUNK), :], wout_buf.at[s],
            wout_sems.at[s],
        )
        co.start()
        return cw, co

    pending = {c: issue_w(c) for c in range(min(SLOTS, N_CHUNKS))}

    barrier_sem = pltpu.get_barrier_semaphore()
    for r in range(1, N_DEV):
        pl.semaphore_signal(
            barrier_sem, inc=1,
            device_id=(jnp.bitwise_xor(my, r),),
            device_id_type=pl.DeviceIdType.MESH,
        )
    pl.semaphore_wait(barrier_sem, N_DEV - 1)

    rs_rdmas = _start_ar(my, p_ref, xb_ref, rs_ref, send_sems, recv_sems)
    acc = _finish_rs(my, rs_rdmas, p_ref, rs_ref)
    _finish_ag(my, acc, x_ref, xb_ref, send_sems, recv_sems)
    x = x_ref[...]

    acc_ref = pacc_ref if scatter else out_ref
    acc_ref[...] = jnp.zeros_like(acc_ref)
    for c in range(N_CHUNKS):
        s = c % SLOTS
        cw, co = pending.pop(c)
        cw.wait()
        co.wait()
        h = jnp.dot(x, win_buf[s], preferred_element_type=jnp.float32)
        h = jnp.maximum(h, 0.0)
        acc_ref[...] += jnp.dot(h, wout_buf[s], preferred_element_type=jnp.float32)
        if c + SLOTS < N_CHUNKS:
            pending[c + SLOTS] = issue_w(c + SLOTS)

    if scatter:
        rs2 = _start_ar(
            my, acc_ref, xb_ref, rs2_ref, send_sems, recv_sems,
            base=2 * (N_DEV - 1),
        )
        out_ref[...] = _finish_rs(my, rs2, acc_ref, rs2_ref)


def _layer(x, win, wout, *, collective_id, scatter=False):
    out_rows = B // N_DEV if scatter else B
    return pl.pallas_call(
        functools.partial(_layer_body, scatter),
        in_specs=[
            pl.BlockSpec(memory_space=pltpu.VMEM),
            pl.BlockSpec(memory_space=pl.ANY),
            pl.BlockSpec(memory_space=pl.ANY),
        ],
        out_specs=pl.BlockSpec(memory_space=pltpu.VMEM),
        out_shape=jax.ShapeDtypeStruct((out_rows, D), jnp.float32),
        scratch_shapes=[
            pltpu.VMEM((B, D), jnp.float32),
            pltpu.VMEM((B, D), jnp.bfloat16),
            pltpu.VMEM((N_DEV - 1, _R, D), jnp.bfloat16),
            pltpu.VMEM((N_DEV - 1, _R, D), jnp.bfloat16),
            pltpu.VMEM((B, D), jnp.float32),
            pltpu.VMEM((SLOTS, D, H_CHUNK), jnp.float32),
            pltpu.VMEM((SLOTS, H_CHUNK, D), jnp.float32),
            pltpu.SemaphoreType.DMA((3 * (N_DEV - 1),)),
            pltpu.SemaphoreType.DMA((3 * (N_DEV - 1),)),
            pltpu.SemaphoreType.DMA((SLOTS,)),
            pltpu.SemaphoreType.DMA((SLOTS,)),
        ],
        compiler_params=pltpu.CompilerParams(
            collective_id=collective_id,
            vmem_limit_bytes=60 * 1024 * 1024,
        ),
    )(x, win, wout)


def _rs_body(p_ref, out_ref, xb_ref, rs_ref, send_sems, recv_sems):
    my = lax.axis_index("i")
    barrier_sem = pltpu.get_barrier_semaphore()
    for r in range(1, N_DEV):
        pl.semaphore_signal(
            barrier_sem, inc=1,
            device_id=(jnp.bitwise_xor(my, r),),
            device_id_type=pl.DeviceIdType.MESH,
        )
    pl.semaphore_wait(barrier_sem, N_DEV - 1)
    rs_rdmas = _start_ar(my, p_ref, xb_ref, rs_ref, send_sems, recv_sems)
    out_ref[...] = _finish_rs(my, rs_rdmas, p_ref, rs_ref)


def _reduce_scatter(p, *, collective_id):
    return pl.pallas_call(
        _rs_body,
        out_shape=jax.ShapeDtypeStruct((B // N_DEV, D), jnp.float32),
        in_specs=[pl.BlockSpec(memory_space=pltpu.VMEM)],
        out_specs=pl.BlockSpec(memory_space=pltpu.VMEM),
        scratch_shapes=[
            pltpu.VMEM((B, D), jnp.bfloat16),
            pltpu.VMEM((N_DEV - 1, _R, D), jnp.bfloat16),
            pltpu.SemaphoreType.DMA((2 * (N_DEV - 1),)),
            pltpu.SemaphoreType.DMA((2 * (N_DEV - 1),)),
        ],
        compiler_params=pltpu.CompilerParams(collective_id=collective_id),
    )(p)


def kernel(x, Win0, Wout0, Win1, Wout1, Win2, Wout2):
    p0 = _layer0(x, Win0, Wout0)
    p1 = _layer(p0, Win1, Wout1, collective_id=0)
    return _layer(p1, Win2, Wout2, collective_id=1, scatter=True)


# device time: 81625 ns/iter; 1.1297x vs baseline; 1.0031x over previous
import functools

import jax
import jax.numpy as jnp
from jax import lax
from jax.experimental import pallas as pl
from jax.experimental.pallas import tpu as pltpu

N_DEV = 8
B = 64
D = 2048
H_SHARD = 4096
H_CHUNK = 512
N_CHUNKS = H_SHARD // H_CHUNK
SLOTS = 6
_R = B // N_DEV


def _layer0_body(x_ref, win_ref, wout_ref, out_ref, acc_ref):
    c = pl.program_id(0)

    @pl.when(c == 0)
    def _():
        acc_ref[...] = jnp.zeros_like(acc_ref)

    h = jnp.dot(x_ref[...], win_ref[...], preferred_element_type=jnp.float32)
    h = jnp.maximum(h, 0.0)
    acc_ref[...] += jnp.dot(h, wout_ref[...], preferred_element_type=jnp.float32)

    @pl.when(c == pl.num_programs(0) - 1)
    def _():
        out_ref[...] = acc_ref[...]


def _layer0(x, win, wout):
    return pl.pallas_call(
        _layer0_body,
        grid=(N_CHUNKS,),
        in_specs=[
            pl.BlockSpec((B, D), lambda c: (0, 0)),
            pl.BlockSpec((D, H_CHUNK), lambda c: (0, c)),
            pl.BlockSpec((H_CHUNK, D), lambda c: (c, 0)),
        ],
        out_specs=pl.BlockSpec((B, D), lambda c: (0, 0)),
        out_shape=jax.ShapeDtypeStruct((B, D), jnp.float32),
        scratch_shapes=[pltpu.VMEM((B, D), jnp.float32)],
    )(x, win, wout)


def _start_ar(my, src_f32_ref, xb_ref, rs_ref, send_sems, recv_sems, base=0):
    xb_ref[...] = src_f32_ref[...].astype(jnp.bfloat16)
    rdmas = []
    for r in range(1, N_DEV):
        partner = jnp.bitwise_xor(my, r)
        rdma = pltpu.make_async_remote_copy(
            src_ref=xb_ref.at[pl.ds(partner * _R, _R)],
            dst_ref=rs_ref.at[r - 1],
            send_sem=send_sems.at[base + r - 1],
            recv_sem=recv_sems.at[base + r - 1],
            device_id=(partner,),
            device_id_type=pl.DeviceIdType.MESH,
        )
        rdma.start()
        rdmas.append(rdma)
    return rdmas


def _finish_rs(my, rs_rdmas, src_f32_ref, rs_ref):
    for rdma in rs_rdmas:
        rdma.wait()
    acc = src_f32_ref[pl.ds(my * _R, _R), :]
    for r in range(1, N_DEV):
        acc = acc + rs_ref[r - 1].astype(jnp.float32)
    return acc


def _finish_ag(my, acc, x_ref, xb_ref, send_sems, recv_sems):
    xb_ref[pl.ds(my * _R, _R), :] = acc.astype(jnp.bfloat16)
    rdmas = []
    for r in range(1, N_DEV):
        partner = jnp.bitwise_xor(my, r)
        rdma = pltpu.make_async_remote_copy(
            src_ref=xb_ref.at[pl.ds(my * _R, _R)],
            dst_ref=xb_ref.at[pl.ds(my * _R, _R)],
            send_sem=send_sems.at[N_DEV - 1 + r - 1],
            recv_sem=recv_sems.at[N_DEV - 1 + r - 1],
            device_id=(partner,),
            device_id_type=pl.DeviceIdType.MESH,
        )
        rdma.start()
        rdmas.append(rdma)
    for rdma in rdmas:
        rdma.wait()
    x_ref[...] = xb_ref[...].astype(jnp.float32)
    x_ref[pl.ds(my * _R, _R), :] = acc


def _layer_body(scatter, p_ref, win_hbm, wout_hbm, out_ref,
                x_ref, xb_ref, rs_ref, rs2_ref, pacc_ref, win_buf, wout_buf,
                send_sems, recv_sems, win_sems, wout_sems):
    my = lax.axis_index("i")

    def issue_w(c):
        s = c % SLOTS
        cw = pltpu.make_async_copy(
            win_hbm.at[:, pl.ds(c * H_CHUNK, H_CHUNK)], win_buf.at[s],
            win_sems.at[s],
        )
        cw.start()
        co = pltpu.make_async_copy(
            wout_hbm.at[pl.ds(c * H_CHUNK, H_CHUNK), :], wout_buf.at[s],
            wout_sems.at[s],
        )
        co.start()
        return cw, co

    pending = {c: issue_w(c) for c in range(min(SLOTS, N_CHUNKS))}

    barrier_sem = pltpu.get_barrier_semaphore()
    for r in range(1, N_DEV):
        pl.semaphore_signal(
            barrier_sem, inc=1,
            device_id=(jnp.bitwise_xor(my, r),),
            device_id_type=pl.DeviceIdType.MESH,
        )
    pl.semaphore_wait(barrier_sem, N_DEV - 1)

    rs_rdmas = _start_ar(my, p_ref, xb_ref, rs_ref, send_sems, recv_sems)
    acc = _finish_rs(my, rs_rdmas, p_ref, rs_ref)
    _finish_ag(my, acc, x_ref, xb_ref, send_sems, recv_sems)
    x = x_ref[...]

    acc_ref = pacc_ref if scatter else out_ref
    acc_ref[...] = jnp.zeros_like(acc_ref)
    for c in range(N_CHUNKS):
        s = c % SLOTS
        cw, co = pending.pop(c)
        cw.wait()
        co.wait()
        h = jnp.dot(x, win_buf[s], preferred_element_type=jnp.float32)
        h = jnp.maximum(h, 0.0)
        acc_ref[...] += jnp.dot(h, wout_buf[s], preferred_element_type=jnp.float32)
        if c + SLOTS < N_CHUNKS:
            pending[c + SLOTS] = issue_w(c + SLOTS)

    if scatter:
        rs2 = _start_ar(
            my, acc_ref, xb_ref, rs2_ref, send_sems, recv_sems,
            base=2 * (N_DEV - 1),
        )
        out_ref[...] = _finish_rs(my, rs2, acc_ref, rs2_ref)


def _layer(x, win, wout, *, collective_id, scatter=False):
    out_rows = B // N_DEV if scatter else B
    return pl.pallas_call(
        functools.partial(_layer_body, scatter),
        in_specs=[
            pl.BlockSpec(memory_space=pltpu.VMEM),
            pl.BlockSpec(memory_space=pl.ANY),
            pl.BlockSpec(memory_space=pl.ANY),
        ],
        out_specs=pl.BlockSpec(memory_space=pltpu.VMEM),
        out_shape=jax.ShapeDtypeStruct((out_rows, D), jnp.float32),
        scratch_shapes=[
            pltpu.VMEM((B, D), jnp.float32),
            pltpu.VMEM((B, D), jnp.bfloat16),
            pltpu.VMEM((N_DEV - 1, _R, D), jnp.bfloat16),
            pltpu.VMEM((N_DEV - 1, _R, D), jnp.bfloat16),
            pltpu.VMEM((B, D), jnp.float32),
            pltpu.VMEM((SLOTS, D, H_CHUNK), jnp.float32),
            pltpu.VMEM((SLOTS, H_CHUNK, D), jnp.float32),
            pltpu.SemaphoreType.DMA((3 * (N_DEV - 1),)),
            pltpu.SemaphoreType.DMA((3 * (N_DEV - 1),)),
            pltpu.SemaphoreType.DMA((SLOTS,)),
            pltpu.SemaphoreType.DMA((SLOTS,)),
        ],
        compiler_params=pltpu.CompilerParams(
            collective_id=collective_id,
            vmem_limit_bytes=60 * 1024 * 1024,
        ),
    )(x, win, wout)


def kernel(x, Win0, Wout0, Win1, Wout1, Win2, Wout2):
    p0 = _layer0(x, Win0, Wout0)
    p1 = _layer(p0, Win1, Wout1, collective_id=0)
    return _layer(p1, Win2, Wout2, collective_id=1, scatter=True)
